# baseline (speedup 1.0000x reference)
import functools

import jax
import jax.numpy as jnp
from jax import lax
from jax.experimental import pallas as pl
from jax.experimental.pallas import tpu as pltpu
from jax.experimental.pallas import tpu_sc as plsc

N_NODES = 10000
N_EDGES = 320000
D = 128

NC = 2
NS = 16
NW = NC * NS
CHUNK = 128
NCH = 80
NBUF = 2
E_PAD = NW * NCH * CHUNK
C0T = 128
C1T = 32
WINC = 16
C0_TOTAL = C0T * NS
TOT_CH = E_PAD // CHUNK
NP = 10240
RPT = NP // NS
HL = 16

_msg_mesh = plsc.VectorSubcoreMesh(core_axis_name="c", subcore_axis_name="s")


@functools.partial(
    pl.kernel,
    out_type=jax.ShapeDtypeStruct((NC, NP, HL), jnp.float32),
    mesh=_msg_mesh,
    scratch_types=[
        pltpu.VMEM((NCH, CHUNK), jnp.int32),
        pltpu.VMEM((CHUNK, HL), jnp.float32),
        pltpu.VMEM((CHUNK, HL), jnp.float32),
        pltpu.VMEM_SHARED((NP, HL), jnp.float32),
    ],
)
def _hist(dst_hbm, out_hbm, dst_v, ones_v, zero_v, acc):
    c = lax.axis_index("c")
    s = lax.axis_index("s")
    w = s * NC + c

    pltpu.sync_copy(dst_hbm.at[w], dst_v)

    one = jnp.ones((16,), jnp.float32)
    zero = jnp.zeros((16,), jnp.float32)

    @pl.loop(0, CHUNK)
    def _fill(i):
        ones_v[i, :] = one
        zero_v[i, :] = zero

    @pl.loop(0, RPT // CHUNK)
    def _zero(t):
        pltpu.sync_copy(zero_v, acc.at[pl.ds(s * RPT + t * CHUNK, CHUNK)])

    plsc.subcore_barrier()

    @pl.loop(0, NCH)
    def _go(j):
        pltpu.sync_copy(ones_v, acc.at[dst_v.at[j]], add=True)

    plsc.subcore_barrier()
    pltpu.sync_copy(acc.at[pl.ds(s * RPT, RPT)], out_hbm.at[c, pl.ds(s * RPT, RPT)])


@functools.partial(
    pl.kernel,
    out_type=jax.ShapeDtypeStruct((NC, NP, D), jnp.float32),
    mesh=_msg_mesh,
    scratch_types=[
        pltpu.VMEM((WINC, CHUNK), jnp.int32),
        pltpu.VMEM((WINC, CHUNK), jnp.int32),
        [pltpu.VMEM((CHUNK, D), jnp.float32) for _ in range(NBUF)],
        [pltpu.SemaphoreType.DMA for _ in range(NBUF)],
        pltpu.VMEM_SHARED((NP, D), jnp.float32),
    ],
)
def _msg(h_hbm, src_hbm, dst_hbm, out_hbm, src_v, dst_v, bufs, sems, acc):
    c = lax.axis_index("c")
    s = lax.axis_index("s")
    nwin = C0T // WINC + c * (C1T // WINC - C0T // WINC)
    base = s * C0T + c * (C0_TOTAL + s * (C1T - C0T))

    zero = jnp.zeros((16,), jnp.float32)
    buf0 = bufs[0]

    @pl.loop(0, CHUNK)
    def _fill(i):
        for k in range(D // 16):
            buf0[i, pl.ds(k * 16, 16)] = zero

    @pl.loop(0, RPT // CHUNK)
    def _zero(t):
        pltpu.sync_copy(buf0, acc.at[pl.ds(s * RPT + t * CHUNK, CHUNK)])

    plsc.subcore_barrier()

    for win in range(C0T // WINC):
        @pl.when(win < nwin)
        def _win(win=win):
            pltpu.sync_copy(src_hbm.at[pl.ds(base + win * WINC, WINC)], src_v)
            pltpu.sync_copy(dst_hbm.at[pl.ds(base + win * WINC, WINC)], dst_v)
            for b in range(NBUF):
                pltpu.async_copy(h_hbm.at[src_v.at[b]], bufs[b], sems[b])

            @pl.loop(0, WINC, step=NBUF)
            def _go(j):
                for b in range(NBUF):
                    pltpu.make_async_copy(
                        h_hbm.at[src_v.at[j + b]], bufs[b], sems[b]).wait()
                    pltpu.sync_copy(bufs[b], acc.at[dst_v.at[j + b]], add=True)

                    @pl.when(j + b + NBUF < WINC)
                    def _pre():
                        pltpu.async_copy(
                            h_hbm.at[src_v.at[j + b + NBUF]], bufs[b], sems[b])

    plsc.subcore_barrier()
    pltpu.sync_copy(acc.at[pl.ds(s * RPT, RPT)], out_hbm.at[c, pl.ds(s * RPT, RPT)])


ROWS = 1000
GRID = N_NODES // ROWS


def _dis_of(hist_ref):
    deg = 1.0 + hist_ref[0, :, 0:1] + hist_ref[1, :, 0:1]
    return lax.rsqrt(deg)


def _tc_first(hist_ref, x_ref, w_ref, out_ref):
    dis = _dis_of(hist_ref)
    h = jnp.dot(x_ref[...], w_ref[...], preferred_element_type=jnp.float32)
    out_ref[...] = h * dis


def _tc_mid(hist_ref, p_ref, hs_ref, b_ref, w_ref, out_ref):
    dis = _dis_of(hist_ref)
    pre = (p_ref[0] + p_ref[1] + hs_ref[...]) * dis + b_ref[...]
    a = jnp.maximum(pre, 0.0)
    out_ref[...] = jnp.dot(a, w_ref[...], preferred_element_type=jnp.float32) * dis


def _tc_last(hist_ref, p_ref, hs_ref, b_ref, wfc_ref, bfc_ref, out_ref):
    dis = _dis_of(hist_ref)
    pre = (p_ref[0] + p_ref[1] + hs_ref[...]) * dis + b_ref[...]
    a = jnp.maximum(pre, 0.0)
    out_ref[...] = (
        jnp.dot(a, wfc_ref[...], preferred_element_type=jnp.float32) + bfc_ref[...]
    )


_hist_spec = pl.BlockSpec((NC, ROWS, HL), lambda i: (0, i, 0))
_p_spec = pl.BlockSpec((NC, ROWS, D), lambda i: (0, i, 0))
_row_spec = pl.BlockSpec((ROWS, D), lambda i: (i, 0))
_w_spec = pl.BlockSpec((D, D), lambda i: (0, 0))
_b_spec = pl.BlockSpec((1, D), lambda i: (0, 0))


def kernel(x, edge_index, W1, b1, W2, b2, Wfc, bfc):
    src = edge_index[0]
    dst = edge_index[1]
    pad = E_PAD - N_EDGES
    src2 = jnp.concatenate([src, jnp.zeros((pad,), jnp.int32)]).reshape(
        TOT_CH, CHUNK)
    dst2 = jnp.concatenate([dst, jnp.full((pad,), N_NODES, jnp.int32)]).reshape(
        TOT_CH, CHUNK)
    dst3 = dst2.reshape(NW, NCH, CHUNK)

    hist = _hist(dst3)

    h1s = pl.pallas_call(
        _tc_first,
        grid=(GRID,),
        in_specs=[_hist_spec, _row_spec, _w_spec],
        out_specs=_row_spec,
        out_shape=jax.ShapeDtypeStruct((N_NODES, D), jnp.float32),
    )(hist, x, W1)

    p1 = _msg(h1s, src2, dst2)

    h2s = pl.pallas_call(
        _tc_mid,
        grid=(GRID,),
        in_specs=[_hist_spec, _p_spec, _row_spec, _b_spec, _w_spec],
        out_specs=_row_spec,
        out_shape=jax.ShapeDtypeStruct((N_NODES, D), jnp.float32),
    )(hist, p1, h1s, b1.reshape(1, D), W2)

    p2 = _msg(h2s, src2, dst2)

    out = pl.pallas_call(
        _tc_last,
        grid=(GRID,),
        in_specs=[
            _hist_spec,
            _p_spec,
            _row_spec,
            _b_spec,
            pl.BlockSpec((D, 1), lambda i: (0, 0)),
            pl.BlockSpec((1, 1), lambda i: (0, 0)),
        ],
        out_specs=pl.BlockSpec((ROWS, 1), lambda i: (i, 0)),
        out_shape=jax.ShapeDtypeStruct((N_NODES, 1), jnp.float32),
    )(hist, p2, h2s, b2.reshape(1, D), Wfc, bfc.reshape(1, 1))

    return out


# speedup vs baseline: 1.0396x; 1.0396x over previous
import functools

import jax
import jax.numpy as jnp
from jax import lax
from jax.experimental import pallas as pl
from jax.experimental.pallas import tpu as pltpu
from jax.experimental.pallas import tpu_sc as plsc

N_NODES = 10000
N_EDGES = 320000
D = 128

NC = 2
NS = 16
NW = NC * NS
CHUNK = 128
NCH = 80
NBUF = 2
E_PAD = NW * NCH * CHUNK
C0T = 144
C1T = 16
WINC = 8
C0_TOTAL = C0T * NS
TOT_CH = E_PAD // CHUNK
NP = 10240
RPT = NP // NS
HL = 16

_msg_mesh = plsc.VectorSubcoreMesh(core_axis_name="c", subcore_axis_name="s")


@functools.partial(
    pl.kernel,
    out_type=jax.ShapeDtypeStruct((NC, NP, HL), jnp.float32),
    mesh=_msg_mesh,
    scratch_types=[
        pltpu.VMEM((NCH, CHUNK), jnp.int32),
        pltpu.VMEM((CHUNK, HL), jnp.float32),
        pltpu.VMEM((CHUNK, HL), jnp.float32),
        pltpu.VMEM_SHARED((NP, HL), jnp.float32),
    ],
)
def _hist(dst_hbm, out_hbm, dst_v, ones_v, zero_v, acc):
    c = lax.axis_index("c")
    s = lax.axis_index("s")
    w = s * NC + c

    pltpu.sync_copy(dst_hbm.at[w], dst_v)

    one = jnp.ones((16,), jnp.float32)
    zero = jnp.zeros((16,), jnp.float32)

    @pl.loop(0, CHUNK)
    def _fill(i):
        ones_v[i, :] = one
        zero_v[i, :] = zero

    @pl.loop(0, RPT // CHUNK)
    def _zero(t):
        pltpu.sync_copy(zero_v, acc.at[pl.ds(s * RPT + t * CHUNK, CHUNK)])

    plsc.subcore_barrier()

    @pl.loop(0, NCH)
    def _go(j):
        pltpu.sync_copy(ones_v, acc.at[dst_v.at[j]], add=True)

    plsc.subcore_barrier()
    pltpu.sync_copy(acc.at[pl.ds(s * RPT, RPT)], out_hbm.at[c, pl.ds(s * RPT, RPT)])


@functools.partial(
    pl.kernel,
    out_type=jax.ShapeDtypeStruct((NC, NP, D), jnp.float32),
    mesh=_msg_mesh,
    scratch_types=[
        pltpu.VMEM((WINC, CHUNK), jnp.int32),
        pltpu.VMEM((WINC, CHUNK), jnp.int32),
        [pltpu.VMEM((CHUNK, D), jnp.float32) for _ in range(NBUF)],
        [pltpu.SemaphoreType.DMA for _ in range(NBUF)],
        pltpu.VMEM_SHARED((NP, D), jnp.float32),
    ],
)
def _msg(h_hbm, src_hbm, dst_hbm, out_hbm, src_v, dst_v, bufs, sems, acc):
    c = lax.axis_index("c")
    s = lax.axis_index("s")
    nwin = C0T // WINC + c * (C1T // WINC - C0T // WINC)
    base = s * C0T + c * (C0_TOTAL + s * (C1T - C0T))

    zero = jnp.zeros((16,), jnp.float32)
    buf0 = bufs[0]

    @pl.loop(0, CHUNK)
    def _fill(i):
        for k in range(D // 16):
            buf0[i, pl.ds(k * 16, 16)] = zero

    @pl.loop(0, RPT // CHUNK)
    def _zero(t):
        pltpu.sync_copy(buf0, acc.at[pl.ds(s * RPT + t * CHUNK, CHUNK)])

    plsc.subcore_barrier()

    for win in range(C0T // WINC):
        @pl.when(win < nwin)
        def _win(win=win):
            pltpu.sync_copy(src_hbm.at[pl.ds(base + win * WINC, WINC)], src_v)
            pltpu.sync_copy(dst_hbm.at[pl.ds(base + win * WINC, WINC)], dst_v)
            for b in range(NBUF):
                pltpu.async_copy(h_hbm.at[src_v.at[b]], bufs[b], sems[b])

            @pl.loop(0, WINC, step=NBUF)
            def _go(j):
                for b in range(NBUF):
                    pltpu.make_async_copy(
                        h_hbm.at[src_v.at[j + b]], bufs[b], sems[b]).wait()
                    pltpu.sync_copy(bufs[b], acc.at[dst_v.at[j + b]], add=True)

                    @pl.when(j + b + NBUF < WINC)
                    def _pre():
                        pltpu.async_copy(
                            h_hbm.at[src_v.at[j + b + NBUF]], bufs[b], sems[b])

    plsc.subcore_barrier()
    pltpu.sync_copy(acc.at[pl.ds(s * RPT, RPT)], out_hbm.at[c, pl.ds(s * RPT, RPT)])


ROWS = 1000
GRID = N_NODES // ROWS


def _dis_of(hist_ref):
    deg = 1.0 + hist_ref[0, :, 0:1] + hist_ref[1, :, 0:1]
    return lax.rsqrt(deg)


def _tc_first(hist_ref, x_ref, w_ref, out_ref):
    dis = _dis_of(hist_ref)
    h = jnp.dot(x_ref[...], w_ref[...], preferred_element_type=jnp.float32)
    out_ref[...] = h * dis


def _tc_mid(hist_ref, p_ref, hs_ref, b_ref, w_ref, out_ref):
    dis = _dis_of(hist_ref)
    pre = (p_ref[0] + p_ref[1] + hs_ref[...]) * dis + b_ref[...]
    a = jnp.maximum(pre, 0.0)
    out_ref[...] = jnp.dot(a, w_ref[...], preferred_element_type=jnp.float32) * dis


def _tc_last(hist_ref, p_ref, hs_ref, b_ref, wfc_ref, bfc_ref, out_ref):
    dis = _dis_of(hist_ref)
    pre = (p_ref[0] + p_ref[1] + hs_ref[...]) * dis + b_ref[...]
    a = jnp.maximum(pre, 0.0)
    out_ref[...] = (
        jnp.dot(a, wfc_ref[...], preferred_element_type=jnp.float32) + bfc_ref[...]
    )


_hist_spec = pl.BlockSpec((NC, ROWS, HL), lambda i: (0, i, 0))
_p_spec = pl.BlockSpec((NC, ROWS, D), lambda i: (0, i, 0))
_row_spec = pl.BlockSpec((ROWS, D), lambda i: (i, 0))
_w_spec = pl.BlockSpec((D, D), lambda i: (0, 0))
_b_spec = pl.BlockSpec((1, D), lambda i: (0, 0))


def kernel(x, edge_index, W1, b1, W2, b2, Wfc, bfc):
    src = edge_index[0]
    dst = edge_index[1]
    pad = E_PAD - N_EDGES
    src2 = jnp.concatenate([src, jnp.zeros((pad,), jnp.int32)]).reshape(
        TOT_CH, CHUNK)
    dst2 = jnp.concatenate([dst, jnp.full((pad,), N_NODES, jnp.int32)]).reshape(
        TOT_CH, CHUNK)
    dst3 = dst2.reshape(NW, NCH, CHUNK)

    hist = _hist(dst3)

    h1s = pl.pallas_call(
        _tc_first,
        grid=(GRID,),
        in_specs=[_hist_spec, _row_spec, _w_spec],
        out_specs=_row_spec,
        out_shape=jax.ShapeDtypeStruct((N_NODES, D), jnp.float32),
    )(hist, x, W1)

    p1 = _msg(h1s, src2, dst2)

    h2s = pl.pallas_call(
        _tc_mid,
        grid=(GRID,),
        in_specs=[_hist_spec, _p_spec, _row_spec, _b_spec, _w_spec],
        out_specs=_row_spec,
        out_shape=jax.ShapeDtypeStruct((N_NODES, D), jnp.float32),
    )(hist, p1, h1s, b1.reshape(1, D), W2)

    p2 = _msg(h2s, src2, dst2)

    out = pl.pallas_call(
        _tc_last,
        grid=(GRID,),
        in_specs=[
            _hist_spec,
            _p_spec,
            _row_spec,
            _b_spec,
            pl.BlockSpec((D, 1), lambda i: (0, 0)),
            pl.BlockSpec((1, 1), lambda i: (0, 0)),
        ],
        out_specs=pl.BlockSpec((ROWS, 1), lambda i: (i, 0)),
        out_shape=jax.ShapeDtypeStruct((N_NODES, 1), jnp.float32),
    )(hist, p2, h2s, b2.reshape(1, D), Wfc, bfc.reshape(1, 1))

    return out
